# Initial kernel scaffold; baseline (speedup 1.0000x reference)
import jax
import jax.numpy as jnp
from jax.experimental import pallas as pl


def kernel(x, edge_index, edge_attr, edge_emb, eps, W1, b1, gamma, beta, W2, b2):
    raise NotImplementedError("write your pallas kernel here")


# speedup vs baseline: 4.4617x; 4.4617x over previous
import functools

import jax
import jax.numpy as jnp
from jax import lax
from jax.experimental import pallas as pl
from jax.experimental.pallas import tpu as pltpu
from jax.experimental.pallas import tpu_sc as plsc

N = 10000
E = 320000
D = 128
K = 4

NC = 2
NS = 16
NW = NC * NS
C = 80
NCHUNK = 128
EP = NW * NCHUNK * C
GB = 16
NGROUP = NCHUNK // GB
NP = 10240
STRIPE = NP // NS
TRASH = N + 64


_YBN = 1000


def _y_body(x_ref, emb_ref, y_ref):
    y_ref[...] = jnp.maximum(x_ref[...][None, :, :] + emb_ref[...][:, None, :], 0.0)


def _build_y(x, edge_emb):
    return pl.pallas_call(
        _y_body,
        grid=(N // _YBN,),
        in_specs=[
            pl.BlockSpec((_YBN, D), lambda r: (r, 0)),
            pl.BlockSpec((K, D), lambda r: (0, 0)),
        ],
        out_specs=pl.BlockSpec((K, _YBN, D), lambda r: (0, r, 0)),
        out_shape=jax.ShapeDtypeStruct((K, N, D), jnp.float32),
    )(x, edge_emb)


def _gidx_body(src_ref, attr_ref, o_ref):
    o_ref[...] = attr_ref[...] * N + src_ref[...]


def _build_gidx(src2, attr2):
    r, c = src2.shape
    return pl.pallas_call(
        _gidx_body,
        out_shape=jax.ShapeDtypeStruct((r, c), jnp.int32),
    )(src2, attr2)


def _sc_body(y_hbm, gidx_hbm, dst_hbm, out_hbm,
             gidx_v, dst_v, rows_a, rows_b, accum):
    cid = lax.axis_index("c")
    tid = lax.axis_index("s")
    wid = cid * NS + tid

    @pl.loop(0, C)
    def _zfill(r):
        for cc in range(D // 16):
            rows_a[r, pl.ds(cc * 16, 16)] = jnp.zeros((16,), jnp.float32)

    for q in range(STRIPE // C):
        pltpu.sync_copy(rows_a, accum.at[pl.ds(tid * STRIPE + q * C, C)])

    plsc.subcore_barrier()

    @pl.loop(0, NGROUP)
    def _group(g):
        pltpu.sync_copy(gidx_hbm.at[wid, pl.ds(g * GB, GB)], gidx_v)
        pltpu.sync_copy(dst_hbm.at[wid, pl.ds(g * GB, GB)], dst_v)

        @pl.loop(0, GB)
        def _chunk(j):
            pltpu.sync_copy(y_hbm.at[gidx_v.at[j]], rows_a)
            pltpu.sync_copy(rows_a, accum.at[dst_v.at[j]], add=True)

    plsc.subcore_barrier()

    pltpu.sync_copy(
        accum.at[pl.ds(tid * STRIPE, STRIPE)],
        out_hbm.at[cid, pl.ds(tid * STRIPE, STRIPE)],
    )


def _sc_segment(y2, gidx_r, dst_r):
    mesh = plsc.VectorSubcoreMesh(core_axis_name="c", subcore_axis_name="s")
    kern = pl.kernel(
        _sc_body,
        out_type=jax.ShapeDtypeStruct((NC, NP, D), jnp.float32),
        mesh=mesh,
        scratch_types=[
            pltpu.VMEM((GB, C), jnp.int32),
            pltpu.VMEM((GB, C), jnp.int32),
            pltpu.VMEM((C, D), jnp.float32),
            pltpu.VMEM((C, D), jnp.float32),
            pltpu.VMEM_SHARED((NP, D), jnp.float32),
        ],
    )
    return kern(y2, gidx_r, dst_r)


_MBN = 1000

_DOT = functools.partial(
    lax.dot_general,
    dimension_numbers=(((1,), (0,)), ((), ())),
    precision=lax.Precision.HIGHEST,
    preferred_element_type=jnp.float32,
)


def _mlp_body(eps_ref, x_ref, p_ref, w1_ref, b1_ref, g_ref, bt_ref, w2_ref,
              b2_ref, o_ref):
    h = (1.0 + eps_ref[0, 0]) * x_ref[...] + p_ref[0] + p_ref[1]
    h = _DOT(h, w1_ref[...]) + b1_ref[...]
    mu = jnp.mean(h, axis=-1, keepdims=True)
    var = jnp.mean((h - mu) * (h - mu), axis=-1, keepdims=True)
    h = (h - mu) / jnp.sqrt(var + 1e-5) * g_ref[...] + bt_ref[...]
    h = jnp.maximum(h, 0.0)
    o_ref[...] = _DOT(h, w2_ref[...]) + b2_ref[...]


def _mlp(x, partials, eps, W1, b1, gamma, beta, W2, b2):
    vec = pl.BlockSpec((1, D), lambda r: (0, 0))
    return pl.pallas_call(
        _mlp_body,
        grid=(N // _MBN,),
        in_specs=[
            pl.BlockSpec((1, 1), lambda r: (0, 0)),
            pl.BlockSpec((_MBN, D), lambda r: (r, 0)),
            pl.BlockSpec((NC, _MBN, D), lambda r: (0, r, 0)),
            pl.BlockSpec((D, D), lambda r: (0, 0)),
            vec, vec, vec,
            pl.BlockSpec((D, D), lambda r: (0, 0)),
            vec,
        ],
        out_specs=pl.BlockSpec((_MBN, D), lambda r: (r, 0)),
        out_shape=jax.ShapeDtypeStruct((N, D), jnp.float32),
    )(eps.reshape(1, 1), x, partials, W1, b1.reshape(1, D), gamma.reshape(1, D),
      beta.reshape(1, D), W2, b2.reshape(1, D))


def kernel(x, edge_index, edge_attr, edge_emb, eps, W1, b1, gamma, beta, W2, b2):
    src = edge_index[0].astype(jnp.int32)
    dst = edge_index[1].astype(jnp.int32)
    attr = edge_attr.astype(jnp.int32)

    gidx = _build_gidx(src.reshape(E // D, D), attr.reshape(E // D, D))
    gidx_r = jnp.concatenate(
        [gidx.reshape(E), jnp.zeros((EP - E,), jnp.int32)]).reshape(NW, NCHUNK, C)
    dst_r = jnp.concatenate(
        [dst, jnp.full((EP - E,), TRASH, jnp.int32)]).reshape(NW, NCHUNK, C)

    y = _build_y(x, edge_emb).reshape(K * N, D)
    partials = _sc_segment(y, gidx_r, dst_r)
    return _mlp(x, partials, eps, W1, b1, gamma, beta, W2, b2)
